# baseline (speedup 1.0000x reference)
import jax
import jax.numpy as jnp
from jax import lax
from jax.experimental import pallas as pl
from jax.experimental.pallas import tpu as pltpu

_NQ = 8
_K = 1024
_D = 8
_N = 16 * 2048
_BLK = 8192
_GRID = _N // _BLK
_COMMIT_SCALE = 0.25 / (_N * _D)


def _rvq_body(xT_ref, cb_ref, cbT_ref, ps_ref, pb_ref, cw_ref, cbias_ref,
              outT_ref, codes_ref, loss_ref):
    b = pl.program_id(0)
    rT = xT_ref[...]
    qsumT = jnp.zeros((_D, _BLK), jnp.float32)
    loss = jnp.float32(0.0)
    iota = lax.broadcasted_iota(jnp.int32, (_K, _BLK), 0)
    for q in range(_NQ):
        cb_q = cb_ref[q]
        cbT_q = cbT_ref[q]
        cbsq = jnp.sum(cb_q * cb_q, axis=1, keepdims=True)
        ch = cbsq.astype(jnp.bfloat16).astype(jnp.float32)
        c1 = cbsq - ch
        cm = c1.astype(jnp.bfloat16).astype(jnp.float32)
        cl = c1 - cm
        A = jnp.concatenate([cb_q * jnp.float32(-2.0), ch, cm, cl], axis=1)
        Bm = jnp.concatenate([rT, jnp.ones((3, _BLK), jnp.float32)], axis=0)
        scores = jnp.dot(A, Bm, preferred_element_type=jnp.float32)
        idx = jnp.argmin(scores, axis=0).reshape(1, _BLK)
        codes_ref[q:q + 1, :] = idx
        onehot = (iota == idx).astype(jnp.bfloat16)
        chi = cbT_q.astype(jnp.bfloat16)
        r1 = cbT_q - chi.astype(jnp.float32)
        cmid = r1.astype(jnp.bfloat16)
        clo = (r1 - cmid.astype(jnp.float32)).astype(jnp.bfloat16)
        cstack = jnp.concatenate([chi, cmid, clo], axis=0)
        emb3 = jnp.dot(cstack, onehot,
                       preferred_element_type=jnp.float32)
        embT = (emb3[0:_D] + emb3[_D:2 * _D]) + emb3[2 * _D:3 * _D]
        s = ps_ref[q:q + 1, :]
        t = pb_ref[q:q + 1, :]
        qsumT = qsumT + (embT * s + t)
        rT = rT - embT
        diff = rT - embT
        loss = loss + jnp.sum(diff * diff)
    outT = jnp.dot(cw_ref[...], qsumT,
                   preferred_element_type=jnp.float32,
                   precision=jax.lax.Precision.HIGHEST) + cbias_ref[...]
    outT_ref[...] = outT
    prev = jnp.where(b == 0, jnp.zeros((1, 1), jnp.float32), loss_ref[...])
    loss_ref[...] = prev + loss * _COMMIT_SCALE


def kernel(x, codebooks, post_scale, post_bias, conv_w, conv_b):
    B, T, D = x.shape
    xT = x.reshape(-1, D).T
    cbT = codebooks.transpose(0, 2, 1)
    ps = post_scale.reshape(_NQ, 1)
    pb = post_bias.reshape(_NQ, 1)
    cbias = conv_b.reshape(D, 1)
    outT, codes, loss = pl.pallas_call(
        _rvq_body,
        grid=(_GRID,),
        in_specs=[
            pl.BlockSpec((_D, _BLK), lambda i: (0, i)),
            pl.BlockSpec((_NQ, _K, _D), lambda i: (0, 0, 0)),
            pl.BlockSpec((_NQ, _D, _K), lambda i: (0, 0, 0)),
            pl.BlockSpec((_NQ, 1), lambda i: (0, 0)),
            pl.BlockSpec((_NQ, 1), lambda i: (0, 0)),
            pl.BlockSpec((_D, _D), lambda i: (0, 0)),
            pl.BlockSpec((_D, 1), lambda i: (0, 0)),
        ],
        out_specs=[
            pl.BlockSpec((_D, _BLK), lambda i: (0, i)),
            pl.BlockSpec((_NQ, _BLK), lambda i: (0, i)),
            pl.BlockSpec((1, 1), lambda i: (0, 0)),
        ],
        out_shape=[
            jax.ShapeDtypeStruct((_D, _N), jnp.float32),
            jax.ShapeDtypeStruct((_NQ, _N), jnp.int32),
            jax.ShapeDtypeStruct((1, 1), jnp.float32),
        ],
        interpret=False,
    )(xT, codebooks, cbT, ps, pb, conv_w, cbias)
    quantized = outT.T.reshape(B, T, D)
    return quantized, loss[0, 0], codes.reshape(_NQ, B, T)


# speedup vs baseline: 1.2115x; 1.2115x over previous
import jax
import jax.numpy as jnp
from jax import lax
from jax.experimental import pallas as pl
from jax.experimental.pallas import tpu as pltpu

_NQ = 8
_K = 1024
_D = 8
_N = 16 * 2048
_BLK = 4096
_GRID = _N // _BLK
_COMMIT_SCALE = 0.25 / (_N * _D)


def _rvq_body(xT_ref, cb_ref, cbT_ref, ps_ref, pb_ref, cw_ref, cbias_ref,
              outT_ref, codes_ref, loss_ref):
    b = pl.program_id(0)
    rT = xT_ref[...]
    qsumT = jnp.zeros((_D, _BLK), jnp.float32)
    loss = jnp.float32(0.0)
    iota = lax.broadcasted_iota(jnp.int32, (_K, _BLK), 0)
    for q in range(_NQ):
        cb_q = cb_ref[q]
        cbT_q = cbT_ref[q]
        cbsq = jnp.sum(cb_q * cb_q, axis=1, keepdims=True)
        ch = cbsq.astype(jnp.bfloat16).astype(jnp.float32)
        c1 = cbsq - ch
        cm = c1.astype(jnp.bfloat16).astype(jnp.float32)
        cl = c1 - cm
        A = jnp.concatenate([cb_q * jnp.float32(-2.0), ch, cm, cl], axis=1)
        Bm = jnp.concatenate([rT, jnp.ones((3, _BLK), jnp.float32)], axis=0)
        scores = jnp.dot(A, Bm, preferred_element_type=jnp.float32)
        idx = jnp.argmin(scores, axis=0).reshape(1, _BLK)
        codes_ref[q:q + 1, :] = idx
        onehot = (iota == idx).astype(jnp.bfloat16)
        chi = cbT_q.astype(jnp.bfloat16)
        r1 = cbT_q - chi.astype(jnp.float32)
        cmid = r1.astype(jnp.bfloat16)
        clo = (r1 - cmid.astype(jnp.float32)).astype(jnp.bfloat16)
        cstack = jnp.concatenate([chi, cmid, clo], axis=0)
        emb3 = jnp.dot(cstack, onehot,
                       preferred_element_type=jnp.float32)
        embT = (emb3[0:_D] + emb3[_D:2 * _D]) + emb3[2 * _D:3 * _D]
        s = ps_ref[q:q + 1, :]
        t = pb_ref[q:q + 1, :]
        qsumT = qsumT + (embT * s + t)
        rT = rT - embT
        diff = rT - embT
        loss = loss + jnp.sum(diff * diff)
    outT = jnp.dot(cw_ref[...], qsumT,
                   preferred_element_type=jnp.float32,
                   precision=jax.lax.Precision.HIGHEST) + cbias_ref[...]
    outT_ref[...] = outT
    prev = jnp.where(b == 0, jnp.zeros((1, 1), jnp.float32), loss_ref[...])
    loss_ref[...] = prev + loss * _COMMIT_SCALE


def kernel(x, codebooks, post_scale, post_bias, conv_w, conv_b):
    B, T, D = x.shape
    xT = x.reshape(-1, D).T
    cbT = codebooks.transpose(0, 2, 1)
    ps = post_scale.reshape(_NQ, 1)
    pb = post_bias.reshape(_NQ, 1)
    cbias = conv_b.reshape(D, 1)
    outT, codes, loss = pl.pallas_call(
        _rvq_body,
        grid=(_GRID,),
        in_specs=[
            pl.BlockSpec((_D, _BLK), lambda i: (0, i)),
            pl.BlockSpec((_NQ, _K, _D), lambda i: (0, 0, 0)),
            pl.BlockSpec((_NQ, _D, _K), lambda i: (0, 0, 0)),
            pl.BlockSpec((_NQ, 1), lambda i: (0, 0)),
            pl.BlockSpec((_NQ, 1), lambda i: (0, 0)),
            pl.BlockSpec((_D, _D), lambda i: (0, 0)),
            pl.BlockSpec((_D, 1), lambda i: (0, 0)),
        ],
        out_specs=[
            pl.BlockSpec((_D, _BLK), lambda i: (0, i)),
            pl.BlockSpec((_NQ, _BLK), lambda i: (0, i)),
            pl.BlockSpec((1, 1), lambda i: (0, 0)),
        ],
        out_shape=[
            jax.ShapeDtypeStruct((_D, _N), jnp.float32),
            jax.ShapeDtypeStruct((_NQ, _N), jnp.int32),
            jax.ShapeDtypeStruct((1, 1), jnp.float32),
        ],
        interpret=False,
    )(xT, codebooks, cbT, ps, pb, conv_w, cbias)
    quantized = outT.T.reshape(B, T, D)
    return quantized, loss[0, 0], codes.reshape(_NQ, B, T)
